# Initial kernel scaffold; baseline (speedup 1.0000x reference)
import jax
import jax.numpy as jnp
from jax.experimental import pallas as pl


def kernel(inputs, W):
    raise NotImplementedError("write your pallas kernel here")


# speedup vs baseline: 1.0620x; 1.0620x over previous
import jax
import jax.numpy as jnp
from jax.experimental import pallas as pl
from jax.experimental.pallas import tpu as pltpu

_NUM_EMB = 1024
_D = 256
_CC = 0.25
_BLK = 1024


def _vq_body(flat_ref, w_ref, q_ref, idx_ref, loss_ref, perp_ref,
             counts_ref, sumsq_ref):
    step = pl.program_id(0)
    nsteps = pl.num_programs(0)
    flat = flat_ref[...]
    w = w_ref[...]

    flat_sq = jnp.sum(flat * flat, axis=1, keepdims=True)
    w_sq = jnp.sum(w * w, axis=1)
    mm = jax.lax.dot_general(flat, w, (((1,), (1,)), ((), ())),
                             preferred_element_type=jnp.float32)
    dist = flat_sq + w_sq - 2.0 * mm

    minval = jnp.min(dist, axis=1, keepdims=True)
    iota = jax.lax.broadcasted_iota(jnp.int32, (_BLK, _NUM_EMB), 1)
    idx = jnp.min(jnp.where(dist == minval, iota, _NUM_EMB), axis=1)

    onehot = (iota == idx[:, None]).astype(jnp.float32)
    q = jnp.dot(onehot, w, preferred_element_type=jnp.float32)

    q_ref[...] = flat + (q - flat)
    idx_ref[...] = idx.reshape(idx_ref.shape)

    @pl.when(step == 0)
    def _init():
        counts_ref[...] = jnp.zeros_like(counts_ref)
        sumsq_ref[0] = 0.0

    counts_ref[...] += jnp.sum(onehot, axis=0).reshape(counts_ref.shape)
    diff = q - flat
    sumsq_ref[0] += jnp.sum(diff * diff)

    @pl.when(step == nsteps - 1)
    def _fin():
        n_rows = nsteps * _BLK
        m = sumsq_ref[0] / (n_rows * _D)
        loss_ref[...] = jnp.full((1, 1), m + _CC * m, jnp.float32)
        avg = counts_ref[...] / n_rows
        perp = jnp.exp(-jnp.sum(avg * jnp.log(avg + 1e-10)))
        perp_ref[...] = jnp.full((1, 1), perp, jnp.float32)


def kernel(inputs, W):
    input_shape = inputs.shape
    flat = inputs.reshape(-1, _D)
    n = flat.shape[0]
    grid = n // _BLK

    q, idx2d, loss, perp = pl.pallas_call(
        _vq_body,
        grid=(grid,),
        in_specs=[
            pl.BlockSpec((_BLK, _D), lambda i: (i, 0)),
            pl.BlockSpec((_NUM_EMB, _D), lambda i: (0, 0)),
        ],
        out_specs=[
            pl.BlockSpec((_BLK, _D), lambda i: (i, 0)),
            pl.BlockSpec((_BLK // 128, 128), lambda i: (i, 0)),
            pl.BlockSpec((1, 1), lambda i: (0, 0)),
            pl.BlockSpec((1, 1), lambda i: (0, 0)),
        ],
        out_shape=[
            jax.ShapeDtypeStruct((n, _D), jnp.float32),
            jax.ShapeDtypeStruct((n // 128, 128), jnp.int32),
            jax.ShapeDtypeStruct((1, 1), jnp.float32),
            jax.ShapeDtypeStruct((1, 1), jnp.float32),
        ],
        scratch_shapes=[
            pltpu.VMEM((_NUM_EMB // 128, 128), jnp.float32),
            pltpu.SMEM((1,), jnp.float32),
        ],
    )(flat, W)

    quantized = q.reshape(input_shape[0], -1)
    encoding_indices = idx2d.reshape(input_shape[0], -1)
    return (loss[0, 0], quantized, perp[0, 0], encoding_indices)


# speedup vs baseline: 1.1921x; 1.1225x over previous
import jax
import jax.numpy as jnp
from jax.experimental import pallas as pl
from jax.experimental.pallas import tpu as pltpu

_NUM_EMB = 1024
_D = 256
_CC = 0.25
_BLK = 1024


def _vq_body(flat_ref, w_ref, q_ref, idx_ref, loss_ref, perp_ref,
             wsq_ref, counts_ref, sumsq_ref):
    step = pl.program_id(0)
    nsteps = pl.num_programs(0)
    flat = flat_ref[...]
    w = w_ref[...]

    @pl.when(step == 0)
    def _init():
        wsq_ref[...] = jnp.sum(w * w, axis=1).reshape(1, _NUM_EMB)
        counts_ref[...] = jnp.zeros_like(counts_ref)
        sumsq_ref[0] = 0.0

    flat_sq = jnp.sum(flat * flat, axis=1, keepdims=True)
    mm2 = jax.lax.dot_general(-2.0 * flat, w, (((1,), (1,)), ((), ())),
                              preferred_element_type=jnp.float32)
    dist = (flat_sq + wsq_ref[...]) + mm2

    minval = jnp.min(dist, axis=1, keepdims=True)
    iota = jax.lax.broadcasted_iota(jnp.int32, (_BLK, _NUM_EMB), 1)
    idx = jnp.min(jnp.where(dist == minval, iota, _NUM_EMB), axis=1)

    onehot = (iota == idx[:, None]).astype(jnp.bfloat16)
    q = jnp.dot(onehot, w.astype(jnp.bfloat16),
                preferred_element_type=jnp.float32)

    q_ref[...] = q
    idx_ref[...] = idx.reshape(idx_ref.shape)

    counts_ref[...] += jnp.dot(jnp.ones((8, _BLK), jnp.bfloat16), onehot,
                               preferred_element_type=jnp.float32)
    sumsq_ref[0] += jnp.sum(minval)

    @pl.when(step == nsteps - 1)
    def _fin():
        n_rows = nsteps * _BLK
        m = sumsq_ref[0] / (n_rows * _D)
        loss_ref[...] = jnp.full((1, 1), m + _CC * m, jnp.float32)
        avg = counts_ref[0:1, :] / n_rows
        perp = jnp.exp(-jnp.sum(avg * jnp.log(avg + 1e-10)))
        perp_ref[...] = jnp.full((1, 1), perp, jnp.float32)


def kernel(inputs, W):
    input_shape = inputs.shape
    flat = inputs.reshape(-1, _D)
    n = flat.shape[0]
    grid = n // _BLK

    q, idx2d, loss, perp = pl.pallas_call(
        _vq_body,
        grid=(grid,),
        in_specs=[
            pl.BlockSpec((_BLK, _D), lambda i: (i, 0)),
            pl.BlockSpec((_NUM_EMB, _D), lambda i: (0, 0)),
        ],
        out_specs=[
            pl.BlockSpec((_BLK, _D), lambda i: (i, 0)),
            pl.BlockSpec((_BLK // 128, 128), lambda i: (i, 0)),
            pl.BlockSpec((1, 1), lambda i: (0, 0)),
            pl.BlockSpec((1, 1), lambda i: (0, 0)),
        ],
        out_shape=[
            jax.ShapeDtypeStruct((n, _D), jnp.float32),
            jax.ShapeDtypeStruct((n // 128, 128), jnp.int32),
            jax.ShapeDtypeStruct((1, 1), jnp.float32),
            jax.ShapeDtypeStruct((1, 1), jnp.float32),
        ],
        scratch_shapes=[
            pltpu.VMEM((1, _NUM_EMB), jnp.float32),
            pltpu.VMEM((8, _NUM_EMB), jnp.float32),
            pltpu.SMEM((1,), jnp.float32),
        ],
    )(flat, W)

    quantized = q.reshape(input_shape[0], -1)
    encoding_indices = idx2d.reshape(input_shape[0], -1)
    return (loss[0, 0], quantized, perp[0, 0], encoding_indices)


# speedup vs baseline: 1.2598x; 1.0567x over previous
import jax
import jax.numpy as jnp
from jax.experimental import pallas as pl
from jax.experimental.pallas import tpu as pltpu

_NUM_EMB = 1024
_D = 256
_CC = 0.25
_BLK = 2048


def _vq_body(flat_ref, w_ref, q_ref, idx_ref, loss_ref, perp_ref,
             wsq_ref, counts_ref, sumsq_ref):
    step = pl.program_id(0)
    nsteps = pl.num_programs(0)
    flat = flat_ref[...]
    w = w_ref[...]

    @pl.when(step == 0)
    def _init():
        wsq_ref[...] = jnp.sum(w * w, axis=1).reshape(1, _NUM_EMB)
        counts_ref[...] = jnp.zeros_like(counts_ref)
        sumsq_ref[0] = 0.0

    flat_sq = jnp.sum(flat * flat, axis=1, keepdims=True)
    mm2 = jax.lax.dot_general(-2.0 * flat, w, (((1,), (1,)), ((), ())),
                              preferred_element_type=jnp.float32)
    dist = (flat_sq + wsq_ref[...]) + mm2

    minval = jnp.min(dist, axis=1, keepdims=True)
    iota = jax.lax.broadcasted_iota(jnp.int32, (_BLK, _NUM_EMB), 1)
    idx = jnp.min(jnp.where(dist == minval, iota, _NUM_EMB), axis=1)

    onehot = (iota == idx[:, None]).astype(jnp.bfloat16)
    q = jnp.dot(onehot, w.astype(jnp.bfloat16),
                preferred_element_type=jnp.float32)

    q_ref[...] = q
    idx_ref[...] = idx.reshape(idx_ref.shape)

    counts_ref[...] += jnp.dot(jnp.ones((8, _BLK), jnp.bfloat16), onehot,
                               preferred_element_type=jnp.float32)
    sumsq_ref[0] += jnp.sum(minval)

    @pl.when(step == nsteps - 1)
    def _fin():
        n_rows = nsteps * _BLK
        m = sumsq_ref[0] / (n_rows * _D)
        loss_ref[...] = jnp.full((1, 1), m + _CC * m, jnp.float32)
        avg = counts_ref[0:1, :] / n_rows
        perp = jnp.exp(-jnp.sum(avg * jnp.log(avg + 1e-10)))
        perp_ref[...] = jnp.full((1, 1), perp, jnp.float32)


def kernel(inputs, W):
    input_shape = inputs.shape
    flat = inputs.reshape(-1, _D)
    n = flat.shape[0]
    grid = n // _BLK

    q, idx2d, loss, perp = pl.pallas_call(
        _vq_body,
        grid=(grid,),
        in_specs=[
            pl.BlockSpec((_BLK, _D), lambda i: (i, 0)),
            pl.BlockSpec((_NUM_EMB, _D), lambda i: (0, 0)),
        ],
        out_specs=[
            pl.BlockSpec((_BLK, _D), lambda i: (i, 0)),
            pl.BlockSpec((_BLK // 128, 128), lambda i: (i, 0)),
            pl.BlockSpec((1, 1), lambda i: (0, 0)),
            pl.BlockSpec((1, 1), lambda i: (0, 0)),
        ],
        out_shape=[
            jax.ShapeDtypeStruct((n, _D), jnp.float32),
            jax.ShapeDtypeStruct((n // 128, 128), jnp.int32),
            jax.ShapeDtypeStruct((1, 1), jnp.float32),
            jax.ShapeDtypeStruct((1, 1), jnp.float32),
        ],
        scratch_shapes=[
            pltpu.VMEM((1, _NUM_EMB), jnp.float32),
            pltpu.VMEM((8, _NUM_EMB), jnp.float32),
            pltpu.SMEM((1,), jnp.float32),
        ],
    )(flat, W)

    quantized = q.reshape(input_shape[0], -1)
    encoding_indices = idx2d.reshape(input_shape[0], -1)
    return (loss[0, 0], quantized, perp[0, 0], encoding_indices)
